# baseline (device time: 36581 ns/iter reference)
import functools

import jax
import jax.numpy as jnp
from jax import lax
from jax.experimental import pallas as pl
from jax.experimental.pallas import tpu as pltpu

N_Z = 4
N_PEERS = 5


def kernel(x):
    _, m, n_full = x.shape
    n_chunk = n_full // N_Z
    m_q = m // 4
    bf16 = jnp.bfloat16

    def body(x_ref, out_ref, slab, sbuf, rbuf,
             slab_sem, dsend_sems, drecv_sems, ex_send_sems, ex_recv_sems):
        xi = lax.axis_index("x")
        yi = lax.axis_index("y")
        zi = lax.axis_index("z")
        q = 2 * xi + yi
        row0 = q * m_q

        slab_cp = pltpu.make_async_copy(
            x_ref.at[0, pl.ds(row0, m_q), :], slab, slab_sem)
        slab_cp.start()

        peers = [(xi, yi, (zi + 1) % N_Z),
                 (xi, yi, (zi + 2) % N_Z),
                 (xi, yi, (zi + 3) % N_Z),
                 (xi, 1 - yi, zi),
                 (1 - xi, yi, zi)]

        barrier_sem = pltpu.get_barrier_semaphore()
        for p in peers:
            pl.semaphore_signal(
                barrier_sem, inc=1,
                device_id=p, device_id_type=pl.DeviceIdType.MESH,
            )
        pl.semaphore_wait(barrier_sem, N_PEERS)
        slab_cp.wait()

        def chunk_f32(c):
            return slab[:, pl.ds(c * n_chunk, n_chunk)]

        sends = []
        for j in range(N_Z - 1):
            tz = (zi + 1 + j) % N_Z
            sbuf[j, :, :] = chunk_f32(tz).astype(bf16)
            rdma = pltpu.make_async_remote_copy(
                src_ref=sbuf.at[j],
                dst_ref=rbuf.at[2 - j],
                send_sem=dsend_sems.at[j],
                recv_sem=drecv_sems.at[2 - j],
                device_id=(xi, yi, tz),
                device_id_type=pl.DeviceIdType.MESH,
            )
            rdma.start()
            sends.append(rdma)

        own = chunk_f32(zi)
        for rdma in sends:
            rdma.wait_recv()
        out_ref[pl.ds(row0, m_q), :] = (
            own
            + rbuf[0, :, :].astype(jnp.float32)
            + rbuf[1, :, :].astype(jnp.float32)
            + rbuf[2, :, :].astype(jnp.float32)).astype(bf16)

        for xv in (0, 1):
            for yv in (0, 1):
                @pl.when(jnp.logical_and(xi == xv, yi == yv))
                def _(xv=xv, yv=yv):
                    r0 = (2 * xv + yv) * m_q
                    yex = pltpu.make_async_remote_copy(
                        src_ref=out_ref.at[pl.ds(r0, m_q)],
                        dst_ref=out_ref.at[pl.ds(r0, m_q)],
                        send_sem=ex_send_sems.at[0],
                        recv_sem=ex_recv_sems.at[0],
                        device_id=(xv, 1 - yv, zi),
                        device_id_type=pl.DeviceIdType.MESH,
                    )
                    yex.start()
                    yex.wait()

        for xv in (0, 1):
            @pl.when(xi == xv)
            def _(xv=xv):
                h0 = xv * 2 * m_q
                xex = pltpu.make_async_remote_copy(
                    src_ref=out_ref.at[pl.ds(h0, 2 * m_q)],
                    dst_ref=out_ref.at[pl.ds(h0, 2 * m_q)],
                    send_sem=ex_send_sems.at[1],
                    recv_sem=ex_recv_sems.at[1],
                    device_id=(1 - xv, yi, zi),
                    device_id_type=pl.DeviceIdType.MESH,
                )
                xex.start()
                xex.wait()

        for rdma in sends:
            rdma.wait_send()

        @functools.partial(
            pl.run_scoped, exit_sem=pltpu.SemaphoreType.REGULAR)
        def _(exit_sem):
            for p in peers:
                pl.semaphore_signal(
                    exit_sem, inc=1,
                    device_id=p, device_id_type=pl.DeviceIdType.MESH,
                )
            pl.semaphore_wait(exit_sem, N_PEERS)

    return pl.pallas_call(
        body,
        out_shape=jax.ShapeDtypeStruct((m, n_chunk), bf16),
        in_specs=[pl.BlockSpec(memory_space=pltpu.MemorySpace.HBM)],
        out_specs=pl.BlockSpec(memory_space=pltpu.VMEM),
        scratch_shapes=[
            pltpu.VMEM((m_q, n_full), jnp.float32),
            pltpu.VMEM((N_Z - 1, m_q, n_chunk), bf16),
            pltpu.VMEM((N_Z - 1, m_q, n_chunk), bf16),
            pltpu.SemaphoreType.DMA,
            pltpu.SemaphoreType.DMA((N_Z - 1,)),
            pltpu.SemaphoreType.DMA((N_Z - 1,)),
            pltpu.SemaphoreType.DMA((2,)),
            pltpu.SemaphoreType.DMA((2,)),
        ],
        compiler_params=pltpu.CompilerParams(collective_id=0),
    )(x)


# device time: 32682 ns/iter; 1.1193x vs baseline; 1.1193x over previous
import functools

import jax
import jax.numpy as jnp
from jax import lax
from jax.experimental import pallas as pl
from jax.experimental.pallas import tpu as pltpu

N_Z = 4
N_PEERS = 6


def kernel(x):
    _, m, n_full = x.shape
    n_chunk = n_full // N_Z
    m_q = m // 4
    bf16 = jnp.bfloat16

    def body(x_ref, out_ref, slab, sbuf, rbuf,
             slab_sem, dsend_sems, drecv_sems, ex_send_sems, ex_recv_sems):
        xi = lax.axis_index("x")
        yi = lax.axis_index("y")
        zi = lax.axis_index("z")
        q = 2 * xi + yi
        row0 = q * m_q

        slab_cp = pltpu.make_async_copy(
            x_ref.at[0, pl.ds(row0, m_q), :], slab, slab_sem)
        slab_cp.start()

        peers = [(xi, yi, (zi + 1) % N_Z),
                 (xi, yi, (zi + 2) % N_Z),
                 (xi, yi, (zi + 3) % N_Z),
                 (xi, 1 - yi, zi),
                 (1 - xi, yi, zi),
                 (1 - xi, 1 - yi, zi)]

        barrier_sem = pltpu.get_barrier_semaphore()
        for p in peers:
            pl.semaphore_signal(
                barrier_sem, inc=1,
                device_id=p, device_id_type=pl.DeviceIdType.MESH,
            )
        pl.semaphore_wait(barrier_sem, N_PEERS)
        slab_cp.wait()

        def chunk_f32(c):
            return slab[:, pl.ds(c * n_chunk, n_chunk)]

        sends = []
        for j in range(N_Z - 1):
            tz = (zi + 1 + j) % N_Z
            sbuf[j, :, :] = chunk_f32(tz).astype(bf16)
            rdma = pltpu.make_async_remote_copy(
                src_ref=sbuf.at[j],
                dst_ref=rbuf.at[2 - j],
                send_sem=dsend_sems.at[j],
                recv_sem=drecv_sems.at[2 - j],
                device_id=(xi, yi, tz),
                device_id_type=pl.DeviceIdType.MESH,
            )
            rdma.start()
            sends.append(rdma)

        own = chunk_f32(zi)
        for rdma in sends:
            rdma.wait_recv()
        out_ref[pl.ds(row0, m_q), :] = (
            own
            + rbuf[0, :, :].astype(jnp.float32)
            + rbuf[1, :, :].astype(jnp.float32)
            + rbuf[2, :, :].astype(jnp.float32)).astype(bf16)

        for xv in (0, 1):
            for yv in (0, 1):
                @pl.when(jnp.logical_and(xi == xv, yi == yv))
                def _(xv=xv, yv=yv):
                    r0 = (2 * xv + yv) * m_q
                    xy_peers = [(xv, 1 - yv), (1 - xv, yv), (1 - xv, 1 - yv)]
                    ex_rdmas = []
                    for r, (px, py) in enumerate(xy_peers):
                        ex = pltpu.make_async_remote_copy(
                            src_ref=out_ref.at[pl.ds(r0, m_q)],
                            dst_ref=out_ref.at[pl.ds(r0, m_q)],
                            send_sem=ex_send_sems.at[r],
                            recv_sem=ex_recv_sems.at[r],
                            device_id=(px, py, zi),
                            device_id_type=pl.DeviceIdType.MESH,
                        )
                        ex.start()
                        ex_rdmas.append(ex)
                    for ex in ex_rdmas:
                        ex.wait()

        for rdma in sends:
            rdma.wait_send()

        @functools.partial(
            pl.run_scoped, exit_sem=pltpu.SemaphoreType.REGULAR)
        def _(exit_sem):
            for p in peers:
                pl.semaphore_signal(
                    exit_sem, inc=1,
                    device_id=p, device_id_type=pl.DeviceIdType.MESH,
                )
            pl.semaphore_wait(exit_sem, N_PEERS)

    return pl.pallas_call(
        body,
        out_shape=jax.ShapeDtypeStruct((m, n_chunk), bf16),
        in_specs=[pl.BlockSpec(memory_space=pltpu.MemorySpace.HBM)],
        out_specs=pl.BlockSpec(memory_space=pltpu.VMEM),
        scratch_shapes=[
            pltpu.VMEM((m_q, n_full), jnp.float32),
            pltpu.VMEM((N_Z - 1, m_q, n_chunk), bf16),
            pltpu.VMEM((N_Z - 1, m_q, n_chunk), bf16),
            pltpu.SemaphoreType.DMA,
            pltpu.SemaphoreType.DMA((N_Z - 1,)),
            pltpu.SemaphoreType.DMA((N_Z - 1,)),
            pltpu.SemaphoreType.DMA((3,)),
            pltpu.SemaphoreType.DMA((3,)),
        ],
        compiler_params=pltpu.CompilerParams(collective_id=0),
    )(x)


# device time: 29887 ns/iter; 1.2240x vs baseline; 1.0935x over previous
import functools

import jax
import jax.numpy as jnp
from jax import lax
from jax.experimental import pallas as pl
from jax.experimental.pallas import tpu as pltpu

N_Z = 4
N_PEERS = 6
B = 2


def kernel(x):
    _, m, n_full = x.shape
    n_chunk = n_full // N_Z
    m_q = m // 4
    m_b = m_q // B
    bf16 = jnp.bfloat16

    def body(x_ref, out_ref, slab, sbuf, rbuf,
             slab_sem, dsend_sems, drecv_sems, ex_send_sems, ex_recv_sems):
        xi = lax.axis_index("x")
        yi = lax.axis_index("y")
        zi = lax.axis_index("z")
        q = 2 * xi + yi
        row0 = q * m_q

        slab_cp = pltpu.make_async_copy(
            x_ref.at[0, pl.ds(row0, m_q), :], slab, slab_sem)
        slab_cp.start()

        peers = [(xi, yi, (zi + 1) % N_Z),
                 (xi, yi, (zi + 2) % N_Z),
                 (xi, yi, (zi + 3) % N_Z),
                 (xi, 1 - yi, zi),
                 (1 - xi, yi, zi),
                 (1 - xi, 1 - yi, zi)]

        barrier_sem = pltpu.get_barrier_semaphore()
        for p in peers:
            pl.semaphore_signal(
                barrier_sem, inc=1,
                device_id=p, device_id_type=pl.DeviceIdType.MESH,
            )
        pl.semaphore_wait(barrier_sem, N_PEERS)
        slab_cp.wait()

        sends = []
        for b in range(B):
            blk = []
            for j in range(N_Z - 1):
                tz = (zi + 1 + j) % N_Z
                e = b * 3 + j
                sbuf[e, :, :] = slab[
                    pl.ds(b * m_b, m_b),
                    pl.ds(tz * n_chunk, n_chunk)].astype(bf16)
                rdma = pltpu.make_async_remote_copy(
                    src_ref=sbuf.at[e],
                    dst_ref=rbuf.at[b * 3 + (2 - j)],
                    send_sem=dsend_sems.at[e],
                    recv_sem=drecv_sems.at[b * 3 + (2 - j)],
                    device_id=(xi, yi, tz),
                    device_id_type=pl.DeviceIdType.MESH,
                )
                rdma.start()
                blk.append(rdma)
            sends.append(blk)

        for b in range(B):
            for rdma in sends[b]:
                rdma.wait_recv()
            own = slab[pl.ds(b * m_b, m_b), pl.ds(zi * n_chunk, n_chunk)]
            out_ref[pl.ds(row0 + b * m_b, m_b), :] = (
                own
                + rbuf[b * 3 + 0, :, :].astype(jnp.float32)
                + rbuf[b * 3 + 1, :, :].astype(jnp.float32)
                + rbuf[b * 3 + 2, :, :].astype(jnp.float32)).astype(bf16)

            for xv in (0, 1):
                for yv in (0, 1):
                    @pl.when(jnp.logical_and(xi == xv, yi == yv))
                    def _(xv=xv, yv=yv, b=b):
                        r0b = (2 * xv + yv) * m_q + b * m_b
                        xy_peers = [(xv, 1 - yv), (1 - xv, yv),
                                    (1 - xv, 1 - yv)]
                        for r, (px, py) in enumerate(xy_peers):
                            ex = pltpu.make_async_remote_copy(
                                src_ref=out_ref.at[pl.ds(r0b, m_b)],
                                dst_ref=out_ref.at[pl.ds(r0b, m_b)],
                                send_sem=ex_send_sems.at[b * 3 + r],
                                recv_sem=ex_recv_sems.at[b * 3 + r],
                                device_id=(px, py, zi),
                                device_id_type=pl.DeviceIdType.MESH,
                            )
                            ex.start()

        for e in range(B * 3):
            w = pltpu.make_async_remote_copy(
                src_ref=out_ref.at[pl.ds(0, m_b)],
                dst_ref=out_ref.at[pl.ds(0, m_b)],
                send_sem=ex_send_sems.at[e],
                recv_sem=ex_recv_sems.at[e],
                device_id=(xi, yi, zi),
                device_id_type=pl.DeviceIdType.MESH,
            )
            w.wait_send()
            w.wait_recv()

        for blk in sends:
            for rdma in blk:
                rdma.wait_send()

        @functools.partial(
            pl.run_scoped, exit_sem=pltpu.SemaphoreType.REGULAR)
        def _(exit_sem):
            for p in peers:
                pl.semaphore_signal(
                    exit_sem, inc=1,
                    device_id=p, device_id_type=pl.DeviceIdType.MESH,
                )
            pl.semaphore_wait(exit_sem, N_PEERS)

    return pl.pallas_call(
        body,
        out_shape=jax.ShapeDtypeStruct((m, n_chunk), bf16),
        in_specs=[pl.BlockSpec(memory_space=pltpu.MemorySpace.HBM)],
        out_specs=pl.BlockSpec(memory_space=pltpu.VMEM),
        scratch_shapes=[
            pltpu.VMEM((m_q, n_full), jnp.float32),
            pltpu.VMEM((B * 3, m_b, n_chunk), bf16),
            pltpu.VMEM((B * 3, m_b, n_chunk), bf16),
            pltpu.SemaphoreType.DMA,
            pltpu.SemaphoreType.DMA((B * 3,)),
            pltpu.SemaphoreType.DMA((B * 3,)),
            pltpu.SemaphoreType.DMA((B * 3,)),
            pltpu.SemaphoreType.DMA((B * 3,)),
        ],
        compiler_params=pltpu.CompilerParams(collective_id=0),
    )(x)
